# Initial kernel scaffold; baseline (speedup 1.0000x reference)
import jax
import jax.numpy as jnp
from jax.experimental import pallas as pl


def kernel(x, qkv_w, out_w, proj_l_w, proj_l_b):
    raise NotImplementedError("write your pallas kernel here")


# speedup vs baseline: 1.5385x; 1.5385x over previous
import math

import jax
import jax.numpy as jnp
from jax import lax
from jax.experimental import pallas as pl
from jax.experimental.pallas import tpu as pltpu

_B, _C, _NH, _D = 2, 768, 12, 64
_L = 1024
_BLK = 64
_NBLK = _L // _BLK
_G = 3 * _NH
_SCALE = 1.0 / math.sqrt(_D)

INTERPRET = False


def _bf(t):
    return t.astype(jnp.bfloat16)


def _qkv_body(x_ref, w_ref, o_ref):
    xb = _bf(x_ref[0])
    wb = _bf(w_ref[...])
    o_ref[0, 0] = lax.dot_general(
        xb, wb, (((0,), (1,)), ((), ())), preferred_element_type=jnp.float32)


def _select_body(q_ref, k_ref, i1_ref, i2_ref):
    q = q_ref[0, 0]
    k = k_ref[0, 0]
    r = lax.broadcasted_iota(jnp.int32, (_NBLK, _L), 0)
    c = lax.broadcasted_iota(jnp.int32, (_NBLK, _L), 1) // _BLK
    pool = jnp.where(r == c, 1.0 / _BLK, 0.0).astype(jnp.float32)
    qm = lax.dot_general(pool, q, (((1,), (0,)), ((), ())),
                         preferred_element_type=jnp.float32,
                         precision=lax.Precision.HIGHEST)
    km = lax.dot_general(pool, k, (((1,), (0,)), ((), ())),
                         preferred_element_type=jnp.float32,
                         precision=lax.Precision.HIGHEST)
    scores = lax.dot_general(_bf(km), _bf(qm), (((1,), (1,)), ((), ())),
                             preferred_element_type=jnp.float32) * _SCALE
    ri = lax.broadcasted_iota(jnp.int32, (_NBLK, _NBLK), 0)
    m1 = jnp.max(scores, axis=0, keepdims=True)
    i1 = jnp.min(jnp.where(scores == m1, ri, _NBLK), axis=0, keepdims=True)
    masked = jnp.where(ri == i1, -jnp.inf, scores)
    m2 = jnp.max(masked, axis=0, keepdims=True)
    i2 = jnp.min(jnp.where(masked == m2, ri, _NBLK), axis=0, keepdims=True)
    i1_ref[0] = i1
    i2_ref[0] = i2


def _attn_body(i1_ref, i2_ref, q_ref, k_ref, v_ref, plw_ref, plb_ref, o_ref):
    bh = pl.program_id(0)
    q = q_ref[0, 0]
    k = k_ref[0, 0]
    v = v_ref[0, 0]
    plw = plw_ref[...]
    plb = plb_ref[...]

    def _fsoftmax(t):
        m = jnp.max(t, axis=-1, keepdims=True)
        e = jnp.exp(t - m)
        return e / jnp.sum(e, axis=-1, keepdims=True)

    ck = _fsoftmax(k)
    cq = _fsoftmax(q)
    kv_tot = lax.dot_general(_bf(ck), _bf(v), (((0,), (0,)), ((), ())),
                             preferred_element_type=jnp.float32)
    z_tot = jnp.sum(ck, axis=0, keepdims=True)

    for i in range(_NBLK):
        j1 = i1_ref[bh, 0, i]
        j2 = i2_ref[bh, 0, i]
        qi = q[i * _BLK:(i + 1) * _BLK, :]
        k1 = k_ref[0, 0, pl.ds(j1 * _BLK, _BLK), :]
        k2 = k_ref[0, 0, pl.ds(j2 * _BLK, _BLK), :]
        v1 = v_ref[0, 0, pl.ds(j1 * _BLK, _BLK), :]
        v2 = v_ref[0, 0, pl.ds(j2 * _BLK, _BLK), :]
        ks = jnp.concatenate([k1, k2], axis=0)
        vs = jnp.concatenate([v1, v2], axis=0)
        logits = lax.dot_general(_bf(qi), _bf(ks), (((1,), (1,)), ((), ())),
                                 preferred_element_type=jnp.float32) * _SCALE
        m = jnp.max(logits, axis=-1, keepdims=True)
        p = jnp.exp(logits - m)
        attn = p / jnp.sum(p, axis=-1, keepdims=True)
        o_s = lax.dot_general(_bf(attn), _bf(vs), (((1,), (0,)), ((), ())),
                              preferred_element_type=jnp.float32)
        ck1 = _fsoftmax(k1)
        ck2 = _fsoftmax(k2)
        kv_sel = (lax.dot_general(_bf(ck1), _bf(v1), (((0,), (0,)), ((), ())),
                                  preferred_element_type=jnp.float32)
                  + lax.dot_general(_bf(ck2), _bf(v2), (((0,), (0,)), ((), ())),
                                    preferred_element_type=jnp.float32))
        kv_q = kv_tot - kv_sel
        z_q = z_tot - jnp.sum(ck1, axis=0, keepdims=True) \
                    - jnp.sum(ck2, axis=0, keepdims=True)
        cqi = cq[i * _BLK:(i + 1) * _BLK, :]
        num = lax.dot_general(_bf(cqi), _bf(kv_q), (((1,), (0,)), ((), ())),
                              preferred_element_type=jnp.float32)
        den = jnp.sum(_bf(cqi).astype(jnp.float32)
                      * _bf(z_q).astype(jnp.float32),
                      axis=-1, keepdims=True) + 1e-6
        o_l = num / den
        o_blk = o_s + lax.dot_general(_bf(o_l), _bf(plw), (((1,), (1,)), ((), ())),
                                      preferred_element_type=jnp.float32) + plb
        o_ref[0, 0, i * _BLK:(i + 1) * _BLK, :] = o_blk


def _outproj_body(w_ref, o_ref, y_ref):
    h = pl.program_id(1)
    part = lax.dot_general(_bf(w_ref[0]), _bf(o_ref[0, 0]),
                           (((1,), (1,)), ((), ())),
                           preferred_element_type=jnp.float32)

    @pl.when(h == 0)
    def _():
        y_ref[0] = part

    @pl.when(h != 0)
    def _():
        y_ref[0] += part


def kernel(x, qkv_w, out_w, proj_l_w, proj_l_b):
    b, c, h, w = x.shape
    assert (b, c, h * w) == (_B, _C, _L)
    xf = x.reshape(_B, _C, _L)

    qkvT = pl.pallas_call(
        _qkv_body,
        grid=(_B, _G),
        in_specs=[
            pl.BlockSpec((1, _C, _L), lambda bb, g: (bb, 0, 0)),
            pl.BlockSpec((_D, _C), lambda bb, g: (g, 0)),
        ],
        out_specs=pl.BlockSpec((1, 1, _L, _D), lambda bb, g: (bb, g, 0, 0)),
        out_shape=jax.ShapeDtypeStruct((_B, _G, _L, _D), jnp.float32),
        interpret=INTERPRET,
    )(xf, qkv_w)

    i1, i2 = pl.pallas_call(
        _select_body,
        grid=(_B * _NH,),
        in_specs=[
            pl.BlockSpec((1, 1, _L, _D), lambda bh: (bh // _NH, bh % _NH, 0, 0)),
            pl.BlockSpec((1, 1, _L, _D),
                         lambda bh: (bh // _NH, _NH + bh % _NH, 0, 0)),
        ],
        out_specs=[
            pl.BlockSpec((1, 1, _NBLK), lambda bh: (bh, 0, 0)),
            pl.BlockSpec((1, 1, _NBLK), lambda bh: (bh, 0, 0)),
        ],
        out_shape=[
            jax.ShapeDtypeStruct((_B * _NH, 1, _NBLK), jnp.int32),
            jax.ShapeDtypeStruct((_B * _NH, 1, _NBLK), jnp.int32),
        ],
        interpret=INTERPRET,
    )(qkvT, qkvT)

    o_heads = pl.pallas_call(
        _attn_body,
        grid=(_B * _NH,),
        in_specs=[
            pl.BlockSpec(memory_space=pltpu.SMEM),
            pl.BlockSpec(memory_space=pltpu.SMEM),
            pl.BlockSpec((1, 1, _L, _D), lambda bh: (bh // _NH, bh % _NH, 0, 0)),
            pl.BlockSpec((1, 1, _L, _D),
                         lambda bh: (bh // _NH, _NH + bh % _NH, 0, 0)),
            pl.BlockSpec((1, 1, _L, _D),
                         lambda bh: (bh // _NH, 2 * _NH + bh % _NH, 0, 0)),
            pl.BlockSpec((_D, _D), lambda bh: (0, 0)),
            pl.BlockSpec((1, _D), lambda bh: (0, 0)),
        ],
        out_specs=pl.BlockSpec((1, 1, _L, _D),
                               lambda bh: (bh // _NH, bh % _NH, 0, 0)),
        out_shape=jax.ShapeDtypeStruct((_B, _NH, _L, _D), jnp.float32),
        interpret=INTERPRET,
    )(i1, i2, qkvT, qkvT, qkvT, proj_l_w, proj_l_b.reshape(1, _D))

    out_w3 = out_w.reshape(_C, _NH, _D).transpose(1, 0, 2)

    y = pl.pallas_call(
        _outproj_body,
        grid=(_B, _NH),
        in_specs=[
            pl.BlockSpec((1, _C, _D), lambda bb, hh: (hh, 0, 0)),
            pl.BlockSpec((1, 1, _L, _D), lambda bb, hh: (bb, hh, 0, 0)),
        ],
        out_specs=pl.BlockSpec((1, _C, _L), lambda bb, hh: (bb, 0, 0)),
        out_shape=jax.ShapeDtypeStruct((_B, _C, _L), jnp.float32),
        interpret=INTERPRET,
    )(out_w3, o_heads)

    return y.reshape(_B, _C, h, w)
